# baseline (speedup 1.0000x reference)
import functools
import math

import jax
import jax.numpy as jnp
from jax import lax
from jax.experimental import pallas as pl
from jax.experimental.pallas import tpu as pltpu
from jax.experimental.pallas import tpu_sc as plsc

DIM = 128
L = 64
WIN = 64
STRIDE = 32
N = 16384
SEG = N // L
SPS = 8
TPS = SPS * SEG
NW = SEG // STRIDE
GRID = L // SPS
INV_TEMP = 1.0 / math.sqrt(DIM)
LAST_N = float(SEG - (NW - 1) * STRIDE)

_DOT_T = (((1,), (1,)), ((), ()))


def _k0_body(z_ref, wqT_ref, pw2_ref, cxf_ref, cyf_ref, keepf_ref,
             qp_ref, stat_ref):
    kf = keepf_ref[...]
    n = jnp.float32(N)
    rows = []
    for cref in (cxf_ref, cyf_ref):
        cm = cref[...] * kf
        s = jnp.sum(cm)
        ss = jnp.sum(cm * cm)
        var = jnp.maximum((ss - s * s / n) / (n - 1.0), 0.0)
        rows.append(jnp.full((1, DIM), 1.0, jnp.float32)
                    / (jnp.sqrt(var) + 1e-8))
    stat_ref[...] = jnp.concatenate(rows, 0)
    qp_ref[...] = jnp.dot(jnp.dot(z_ref[...], wqT_ref[...]), pw2_ref[...])


def _ks_body(cxl_hbm, cyl_hbm, keepl_hbm, qp_hbm, posP_hbm, stat_hbm,
             phR_hbm, phL_hbm,
             cx_v, cy_v, kp_v, qp_v, p0_v, p1_v, p2_v, st_v,
             phR_v, phL_v):
    nc = 2
    wid = lax.axis_index("s") * nc + lax.axis_index("c")

    pltpu.sync_copy(posP_hbm.at[pl.ds(0, DIM)], p0_v)
    pltpu.sync_copy(posP_hbm.at[pl.ds(DIM, DIM)], p1_v)
    pltpu.sync_copy(posP_hbm.at[pl.ds(2 * DIM, DIM)], p2_v)
    pltpu.sync_copy(stat_hbm, st_v)
    isx = st_v[pl.ds(0, 16)]
    isy = st_v[pl.ds(DIM, 16)]

    for s_local in range(2):
        seg = wid * 2 + s_local
        pltpu.sync_copy(cxl_hbm.at[pl.ds(seg * SEG, SEG)], cx_v)
        pltpu.sync_copy(cyl_hbm.at[pl.ds(seg * SEG, SEG)], cy_v)
        pltpu.sync_copy(keepl_hbm.at[pl.ds(seg * SEG, SEG)], kp_v)
        pltpu.sync_copy(qp_hbm.at[pl.ds(seg * DIM, DIM)], qp_v)

        lane = jnp.arange(16, dtype=jnp.int32)

        def allsum(v):
            for sh in (8, 4, 2, 1):
                idx = (lane + sh) % 16
                v = v + v.at[idx].get(mode="promise_in_bounds")
            return v

        sx = []
        sy = []
        for c in range(NW):
            sl0 = pl.ds(c * STRIDE, 16)
            sl1 = pl.ds(c * STRIDE + 16, 16)
            sx.append(allsum(cx_v[sl0] * kp_v[sl0]
                             + cx_v[sl1] * kp_v[sl1]))
            sy.append(allsum(cy_v[sl0] * kp_v[sl0]
                             + cy_v[sl1] * kp_v[sl1]))

        mxR = [(sx[c] + sx[c + 1]) / WIN for c in range(NW - 1)]
        mxR.append(sx[NW - 1] / LAST_N)
        myR = [(sy[c] + sy[c + 1]) / WIN for c in range(NW - 1)]
        myR.append(sy[NW - 1] / LAST_N)

        for pair in range(NW // 2):
            cwxR = []
            cwyR = []
            cwxL = []
            cwyL = []
            slices = []
            for c in (2 * pair, 2 * pair + 1):
                cL = max(c - 1, 0)
                for h in range(2):
                    sl = pl.ds(c * STRIDE + h * 16, 16)
                    slices.append(sl)
                    k16 = kp_v[sl]
                    cxm = cx_v[sl] * k16
                    cym = cy_v[sl] * k16
                    cwxR.append((cxm - mxR[c]) * isx)
                    cwyR.append((cym - myR[c]) * isy)
                    cwxL.append((cxm - mxR[cL]) * isx)
                    cwyL.append((cym - myR[cL]) * isy)

            def jstep(j, ph):
                jb = (j // 16) * 16
                idx = jnp.full((16,), j - jb, jnp.int32)

                def bcast(ref):
                    blk = ref[pl.ds(jb, 16)]
                    return blk.at[idx].get(mode="promise_in_bounds")

                p0 = bcast(p0_v)
                p1 = bcast(p1_v)
                p2 = bcast(p2_v)
                qj = bcast(qp_v)
                new = []
                for v in range(4):
                    hR = jnp.maximum(cwxR[v] * p0 + cwyR[v] * p1 + p2,
                                     0.0)
                    hL = jnp.maximum(cwxL[v] * p0 + cwyL[v] * p1 + p2,
                                     0.0)
                    new.append(ph[2 * v] + qj * hR)
                    new.append(ph[2 * v + 1] + qj * hL)
                return tuple(new)

            z16 = jnp.zeros((16,), jnp.float32)
            ph = lax.fori_loop(0, DIM, jstep, (z16,) * 8)
            for v in range(4):
                phR_v[slices[v]] = ph[2 * v]
                phL_v[slices[v]] = ph[2 * v + 1]

        pltpu.sync_copy(phR_v, phR_hbm.at[pl.ds(seg * SEG, SEG)])
        pltpu.sync_copy(phL_v, phL_hbm.at[pl.ds(seg * SEG, SEG)])


def _k2a_body(f_ref, z_ref, wqT_ref, wk_ref, mu_ref, sq_ref, qkf_ref,
              qk_sc):
    i = pl.program_id(0)

    @pl.when(i == 0)
    def _():
        qk_sc[...] = jnp.dot(jnp.dot(z_ref[...], wqT_ref[...]),
                             wk_ref[...])

    f = f_ref[...]
    ones_col = jnp.ones((DIM, 1), jnp.float32)
    mu_ref[...] = jnp.dot(f, ones_col) * (1.0 / DIM)
    sq_ref[...] = jnp.dot(f * f, ones_col) * (1.0 / DIM)
    qk8 = qk_sc[pl.ds(i * SPS, SPS), :]
    qk_exp = jnp.broadcast_to(qk8[:, None, :],
                              (SPS, SEG, DIM)).reshape(TPS, DIM)
    qkf_ref[...] = jnp.dot(f * qk_exp, ones_col)


def _k2b_body(f_ref, keepl_ref, mu_ref, sq_ref, qkf_ref, phR_ref,
              phL_ref, z_ref, wqT_ref, wk_ref, pb2_ref, wvT_ref,
              woT_ref, bo_ref, bown_ref, bnext_ref, bprev_ref, o_ref,
              qb_sc, qs_sc, wacc_sc):
    i = pl.program_id(0)

    @pl.when(i == 0)
    def _():
        q_all = jnp.dot(z_ref[...], wqT_ref[...])
        qk = jnp.dot(q_all, wk_ref[...])
        qb_sc[...] = jnp.dot(q_all, pb2_ref[...])
        qs_sc[...] = jnp.dot(qk, jnp.ones((DIM, 1), jnp.float32))

    keep_l = keepl_ref[...]
    mu = mu_ref[...]
    sq = sq_ref[...]
    qkf = qkf_ref[...]
    rstd = lax.rsqrt(jnp.maximum(sq - mu * mu, 0.0) + 1e-5)
    qb8 = qb_sc[pl.ds(i * SPS, SPS), :]
    qs8 = qs_sc[pl.ds(i * SPS, SPS), :]
    a = rstd * (qkf - mu * qs8) * keep_l

    def elogit(ph):
        return jnp.exp(jnp.clip((a + ph + qb8) * INV_TEMP, -10.0, 10.0))

    eR = elogit(phR_ref[...])
    eL = elogit(phL_ref[...])

    den = jnp.dot(eR, bown_ref[...]) + jnp.dot(eL, bnext_ref[...])
    den_prev = jnp.dot(den, bprev_ref[...])
    lane_chunk = lax.broadcasted_iota(jnp.int32, (SPS, SEG), 1) // STRIDE
    aR = eR / den
    aL = jnp.where(lane_chunk == 0, 0.0, eL / den_prev)
    g = (aR + aL) * keep_l * ((1.0 / NW) * rstd)

    f = f_ref[...]
    w_rows = [jnp.dot(g[s:s + 1, :], f[s * SEG:(s + 1) * SEG, :])
              for s in range(SPS)]
    corr = jnp.dot(g * mu, jnp.ones((SEG, 1), jnp.float32))
    wacc_sc[pl.ds(i * SPS, SPS), :] = jnp.concatenate(w_rows, 0) - corr

    @pl.when(i == GRID - 1)
    def _():
        zv = jnp.dot(wacc_sc[...], wvT_ref[...])
        o_ref[...] = jnp.dot(zv, woT_ref[...]) + bo_ref[...]


def _chunk_mats():
    j = jnp.arange(SEG)[:, None] // STRIDE
    k = jnp.arange(SEG)[None, :] // STRIDE
    bown = (j == k).astype(jnp.float32)
    bnext = (j == k + 1).astype(jnp.float32)
    bprev = jnp.where(j == k - 1, 1.0 / STRIDE, 0.0).astype(jnp.float32)
    return bown, bnext, bprev


@jax.jit
def kernel(feats, coords, mask, z, Wq, Wk, Wv, posW1, posb1, posW2, posb2,
           Wo, bo):
    keep = 1.0 - mask.astype(jnp.float32)
    keepl = keep.reshape(L, SEG)
    cxl = coords[:, 0].reshape(L, SEG)
    cyl = coords[:, 1].reshape(L, SEG)
    posP = jnp.stack([posW1[:, 0], posW1[:, 1], posb1], axis=0)
    bown, bnext, bprev = _chunk_mats()
    full = lambda shape: pl.BlockSpec(shape, lambda i: (0, 0))
    seg = lambda shape: pl.BlockSpec(shape, lambda i: (i, 0))

    qp_all, stats = pl.pallas_call(
        _k0_body,
        out_shape=[jax.ShapeDtypeStruct((L, DIM), jnp.float32),
                   jax.ShapeDtypeStruct((2, DIM), jnp.float32)],
    )(z, Wq.T, posW2, cxl, cyl, keepl)

    mesh = plsc.VectorSubcoreMesh(core_axis_name="c", subcore_axis_name="s")
    ks = functools.partial(
        pl.kernel, mesh=mesh,
        out_type=[jax.ShapeDtypeStruct((N,), jnp.float32),
                  jax.ShapeDtypeStruct((N,), jnp.float32)],
        scratch_types=[
            pltpu.VMEM((SEG,), jnp.float32),
            pltpu.VMEM((SEG,), jnp.float32),
            pltpu.VMEM((SEG,), jnp.float32),
            pltpu.VMEM((DIM,), jnp.float32),
            pltpu.VMEM((DIM,), jnp.float32),
            pltpu.VMEM((DIM,), jnp.float32),
            pltpu.VMEM((DIM,), jnp.float32),
            pltpu.VMEM((2 * DIM,), jnp.float32),
            pltpu.VMEM((SEG,), jnp.float32),
            pltpu.VMEM((SEG,), jnp.float32),
        ])(_ks_body)
    phR_f, phL_f = ks(cxl.reshape(N), cyl.reshape(N), keepl.reshape(N),
                      qp_all.reshape(L * DIM), posP.reshape(3 * DIM),
                      stats.reshape(2 * DIM))
    phR = phR_f.reshape(L, SEG)
    phL = phL_f.reshape(L, SEG)

    mu_c, sq_c, qkf_c = pl.pallas_call(
        _k2a_body,
        grid=(GRID,),
        in_specs=[seg((TPS, DIM)), full((L, DIM)), full((DIM, DIM)),
                  full((DIM, DIM))],
        out_specs=[seg((TPS, 1))] * 3,
        out_shape=[jax.ShapeDtypeStruct((N, 1), jnp.float32)] * 3,
        scratch_shapes=[pltpu.VMEM((L, DIM), jnp.float32)],
    )(feats, z, Wq.T, Wk)

    out = pl.pallas_call(
        _k2b_body,
        grid=(GRID,),
        in_specs=[
            seg((TPS, DIM)),
            seg((SPS, SEG)),
            seg((SPS, SEG)), seg((SPS, SEG)), seg((SPS, SEG)),
            seg((SPS, SEG)), seg((SPS, SEG)),
            full((L, DIM)), full((DIM, DIM)), full((DIM, DIM)),
            full((DIM, 1)),
            full((DIM, DIM)), full((DIM, DIM)), full((1, DIM)),
            full((SEG, SEG)), full((SEG, SEG)), full((SEG, SEG)),
        ],
        out_specs=full((L, DIM)),
        out_shape=jax.ShapeDtypeStruct((L, DIM), jnp.float32),
        scratch_shapes=[
            pltpu.VMEM((L, 1), jnp.float32),
            pltpu.VMEM((L, 1), jnp.float32),
            pltpu.VMEM((L, DIM), jnp.float32),
        ],
    )(feats, keepl, mu_c.reshape(L, SEG), sq_c.reshape(L, SEG),
      qkf_c.reshape(L, SEG), phR, phL, z, Wq.T, Wk,
      posb2.reshape(DIM, 1), Wv.T, Wo.T, bo.reshape(1, DIM),
      bown, bnext, bprev)
    return out
